# baseline (speedup 1.0000x reference)
import functools

import jax
import jax.numpy as jnp
from jax import lax
from jax.experimental import pallas as pl
from jax.experimental.pallas import tpu as pltpu
from jax.experimental.pallas import tpu_sc as plsc

_DEPTH = 1000
_CH = 32


def _onehot_sc_body(x_hbm, out_hbm, idx_v, buf0, buf1, sem0, sem1):
    nc = lax.axis_size("c")
    wid = lax.axis_index("s") * nc + lax.axis_index("c")
    rows_per_w = idx_v.shape[0]
    n_chunks = rows_per_w // _CH
    base_row = wid * rows_per_w

    pltpu.sync_copy(x_hbm.at[pl.ds(base_row, rows_per_w)], idx_v)

    iota16 = lax.broadcasted_iota(jnp.int32, (16,), 0)
    ones_v = jnp.ones((16,), jnp.float32)
    zeros_v = jnp.zeros((16,), jnp.float32)

    def _zero_step(i, _):
        for j in range(8):
            buf0[pl.ds(i * 128 + j * 16, 16)] = zeros_v
            buf1[pl.ds(i * 128 + j * 16, 16)] = zeros_v
        return 0
    lax.fori_loop(0, (_CH * _DEPTH) // 128, _zero_step, 0)

    bufs = (buf0, buf1)
    sems = (sem0, sem1)

    def _flat_positions(c):
        out = []
        for k in range(_CH // 16):
            cols = idx_v[pl.ds(c * _CH + k * 16, 16)]
            out.append((k * 16 + iota16) * _DEPTH + cols)
        return out

    pending = [None, None]
    for c in range(n_chunks):
        b = c % 2
        if pending[b] is not None:
            handle, old_c = pending[b]
            handle.wait()
            for pos in _flat_positions(old_c):
                plsc.store_scatter(bufs[b], [pos], zeros_v)
        for pos in _flat_positions(c):
            plsc.store_scatter(bufs[b], [pos], ones_v)
        dst = out_hbm.at[pl.ds((base_row + c * _CH) * _DEPTH, _CH * _DEPTH)]
        handle = pltpu.async_copy(bufs[b], dst, sems[b])
        pending[b] = (handle, c)
    for b in range(2):
        if pending[b] is not None:
            pending[b][0].wait()


def kernel(X_in, ones):
    del ones
    batch = X_in.shape[0]
    info = plsc.get_sparse_core_info()
    nw = info.num_cores * info.num_subcores
    rows_per_w = batch // nw
    mesh = plsc.VectorSubcoreMesh(core_axis_name="c", subcore_axis_name="s")
    sc_call = pl.kernel(
        _onehot_sc_body,
        out_type=jax.ShapeDtypeStruct((batch * _DEPTH,), jnp.float32),
        mesh=mesh,
        scratch_types=[
            pltpu.VMEM((rows_per_w,), jnp.int32),
            pltpu.VMEM((_CH * _DEPTH,), jnp.float32),
            pltpu.VMEM((_CH * _DEPTH,), jnp.float32),
            pltpu.SemaphoreType.DMA,
            pltpu.SemaphoreType.DMA,
        ],
        compiler_params=pltpu.CompilerParams(needs_layout_passes=False),
    )
    flat = sc_call(X_in.astype(jnp.int32))
    return flat.reshape(batch, _DEPTH)


# speedup vs baseline: 1.6127x; 1.6127x over previous
import jax
import jax.numpy as jnp
from jax import lax
from jax.experimental import pallas as pl
from jax.experimental.pallas import tpu as pltpu
from jax.experimental.pallas import tpu_sc as plsc

_DEPTH = 1000
_CH = 32


def _onehot_sc_body(x_hbm, out_hbm, idx_v, buf0, buf1, sem0, sem1):
    nc = lax.axis_size("c")
    wid = lax.axis_index("s") * nc + lax.axis_index("c")
    rows_per_w = idx_v.shape[0]
    n_chunks = rows_per_w // _CH
    base_row = wid * rows_per_w

    pltpu.sync_copy(x_hbm.at[pl.ds(base_row, rows_per_w)], idx_v)

    iota16 = lax.broadcasted_iota(jnp.int32, (16,), 0)
    ones_v = jnp.ones((16,), jnp.float32)
    zeros_v = jnp.zeros((16,), jnp.float32)

    def _zero_row(r, _):
        for buf in (buf0, buf1):
            for j in range(62):
                buf[r, pl.ds(j * 16, 16)] = zeros_v
            buf[r, pl.ds(984, 16)] = zeros_v
        return 0
    lax.fori_loop(0, _CH, _zero_row, 0)

    bufs = (buf0, buf1)
    sems = (sem0, sem1)

    def _positions(c):
        out = []
        for k in range(_CH // 16):
            cols = idx_v[pl.ds(c * _CH + k * 16, 16)]
            out.append((k * 16 + iota16, cols))
        return out

    pending = [None, None]
    for c in range(n_chunks):
        b = c % 2
        if pending[b] is not None:
            handle, old_c = pending[b]
            handle.wait()
            for rows, cols in _positions(old_c):
                plsc.store_scatter(bufs[b], [rows, cols], zeros_v)
        for rows, cols in _positions(c):
            plsc.store_scatter(bufs[b], [rows, cols], ones_v)
        dst = out_hbm.at[pl.ds(base_row + c * _CH, _CH)]
        handle = pltpu.async_copy(bufs[b], dst, sems[b])
        pending[b] = (handle, c)
    for b in range(2):
        if pending[b] is not None:
            pending[b][0].wait()


def kernel(X_in, ones):
    del ones
    batch = X_in.shape[0]
    info = plsc.get_sparse_core_info()
    nw = info.num_cores * info.num_subcores
    rows_per_w = batch // nw
    mesh = plsc.VectorSubcoreMesh(core_axis_name="c", subcore_axis_name="s")
    sc_call = pl.kernel(
        _onehot_sc_body,
        out_type=jax.ShapeDtypeStruct((batch, _DEPTH), jnp.float32),
        mesh=mesh,
        scratch_types=[
            pltpu.VMEM((rows_per_w,), jnp.int32),
            pltpu.VMEM((_CH, _DEPTH), jnp.float32),
            pltpu.VMEM((_CH, _DEPTH), jnp.float32),
            pltpu.SemaphoreType.DMA,
            pltpu.SemaphoreType.DMA,
        ],
        compiler_params=pltpu.CompilerParams(needs_layout_passes=False),
    )
    return sc_call(X_in.astype(jnp.int32))


# speedup vs baseline: 1.6180x; 1.0033x over previous
import jax
import jax.numpy as jnp
from jax import lax
from jax.experimental import pallas as pl
from jax.experimental.pallas import tpu as pltpu
from jax.experimental.pallas import tpu_sc as plsc

_DEPTH = 1000
_CH = 32


def _onehot_sc_body(x_hbm, out_hbm, idx_v, buf0, buf1, sem0, sem1):
    nc = lax.axis_size("c")
    wid = lax.axis_index("s") * nc + lax.axis_index("c")
    rows_per_w = idx_v.shape[0]
    n_chunks = rows_per_w // _CH
    base_row = wid * rows_per_w

    pltpu.sync_copy(x_hbm.at[pl.ds(base_row, rows_per_w)], idx_v)

    iota16 = lax.broadcasted_iota(jnp.int32, (16,), 0)
    ones_v = jnp.ones((16,), jnp.float32)
    zeros_v = jnp.zeros((16,), jnp.float32)

    def _zero_row(r, _):
        for buf in (buf0, buf1):
            for j in range(62):
                buf[r, pl.ds(j * 16, 16)] = zeros_v
            buf[r, pl.ds(984, 16)] = zeros_v
        return 0
    lax.fori_loop(0, _CH, _zero_row, 0)

    bufs = (buf0, buf1)
    sems = (sem0, sem1)

    def _positions(c):
        out = []
        for k in range(_CH // 16):
            cols = idx_v[pl.ds(c * _CH + k * 16, 16)]
            out.append((k * 16 + iota16, cols))
        return out

    pending = [None, None]
    for c in range(n_chunks):
        b = c % 2
        if pending[b] is not None:
            handle, old_c = pending[b]
            handle.wait()
            for rows, cols in _positions(old_c):
                plsc.store_scatter(bufs[b], [rows, cols], zeros_v)
        for rows, cols in _positions(c):
            plsc.store_scatter(bufs[b], [rows, cols], ones_v)
        dst = out_hbm.at[pl.ds(base_row + c * _CH, _CH)]
        handle = pltpu.async_copy(bufs[b], dst, sems[b])
        pending[b] = (handle, c)
    for b in range(2):
        if pending[b] is not None:
            pending[b][0].wait()


def kernel(X_in, ones):
    del ones
    batch = X_in.shape[0]
    info = plsc.get_sparse_core_info()
    nw = info.num_cores * info.num_subcores
    rows_per_w = batch // nw
    mesh = plsc.VectorSubcoreMesh(core_axis_name="c", subcore_axis_name="s")
    sc_call = pl.kernel(
        _onehot_sc_body,
        out_type=jax.ShapeDtypeStruct((batch, _DEPTH), jnp.float32),
        mesh=mesh,
        scratch_types=[
            pltpu.VMEM((rows_per_w,), jnp.int32),
            pltpu.VMEM((_CH, _DEPTH), jnp.float32),
            pltpu.VMEM((_CH, _DEPTH), jnp.float32),
            pltpu.SemaphoreType.DMA,
            pltpu.SemaphoreType.DMA,
        ],
        compiler_params=pltpu.CompilerParams(
            needs_layout_passes=False, use_tc_tiling_on_sc=True),
    )
    return sc_call(X_in.astype(jnp.int32))


# speedup vs baseline: 6.0305x; 3.7272x over previous
import jax
import jax.numpy as jnp
from jax.experimental import pallas as pl

_BLK = 512
_DEPTH = 1000


def _onehot_t_body(x_ref, out_ref):
    idx = x_ref[0, 0, :]
    depth, blk = out_ref.shape
    rows = jax.lax.broadcasted_iota(jnp.int32, (depth, blk), 0)
    out_ref[...] = (rows == idx[None, :]).astype(jnp.float32)


def kernel(X_in, ones):
    del ones
    batch = X_in.shape[0]
    grid = batch // _BLK
    x3 = X_in.astype(jnp.int32).reshape(grid, 1, _BLK)
    outT = pl.pallas_call(
        _onehot_t_body,
        grid=(grid,),
        in_specs=[pl.BlockSpec((1, 1, _BLK), lambda i: (i, 0, 0))],
        out_specs=pl.BlockSpec((_DEPTH, _BLK), lambda i: (0, i)),
        out_shape=jax.ShapeDtypeStruct((_DEPTH, batch), jnp.float32),
    )(x3)
    return outT.T
